# Initial kernel scaffold; baseline (speedup 1.0000x reference)
import jax
import jax.numpy as jnp
from jax.experimental import pallas as pl


def kernel(ids, char_emb, W, b):
    raise NotImplementedError("write your pallas kernel here")


# speedup vs baseline: 87.7153x; 87.7153x over previous
import functools

import jax
import jax.numpy as jnp
from jax import lax
from jax.experimental import pallas as pl
from jax.experimental.pallas import tpu as pltpu
from jax.experimental.pallas import tpu_sc as plsc

_NC, _NS = 2, 16
_NW = _NC * _NS
_LANES = 16


def _sc_row_histogram(ids, vocab):
    B, S = ids.shape
    rows_per_w = B // _NW
    groups = rows_per_w // _LANES
    mesh = plsc.VectorSubcoreMesh(core_axis_name="c", subcore_axis_name="s")

    @functools.partial(
        pl.kernel,
        mesh=mesh,
        out_type=jax.ShapeDtypeStruct((B, vocab), jnp.float32),
        scratch_types=[
            pltpu.VMEM((_LANES, S), jnp.int32),
            pltpu.VMEM((_LANES, vocab), jnp.float32),
        ],
        compiler_params=pltpu.CompilerParams(
            use_tc_tiling_on_sc=False, needs_layout_passes=False
        ),
    )
    def hist_kernel(ids_hbm, counts_hbm, ids_v, hist_v):
        wid = lax.axis_index("s") * _NC + lax.axis_index("c")
        lane = lax.iota(jnp.int32, _LANES)
        ones = jnp.ones((_LANES,), jnp.float32)
        zeros = jnp.zeros((_LANES,), jnp.float32)

        def group_body(g, carry):
            base = wid * rows_per_w + g * _LANES
            pltpu.sync_copy(ids_hbm.at[pl.ds(base, _LANES), :], ids_v)
            for i in range(_LANES):
                for c in range(vocab // _LANES):
                    hist_v[i, pl.ds(c * _LANES, _LANES)] = zeros

            def char_body(j, c2):
                jv = jnp.full((_LANES,), j, dtype=jnp.int32)
                idv = plsc.load_gather(ids_v, [lane, jv])
                plsc.addupdate_scatter(hist_v, [lane, idv], ones)
                return c2

            lax.fori_loop(0, S, char_body, 0)
            pltpu.sync_copy(hist_v, counts_hbm.at[pl.ds(base, _LANES), :])
            return carry

        lax.fori_loop(0, groups, group_body, 0)

    return hist_kernel(ids)


def _tc_project(counts, char_emb, W, b2d, inv_n):
    B, V = counts.shape
    E = char_emb.shape[1]
    T = W.shape[1]
    BT = 1024
    grid = B // BT

    def body(counts_ref, emb_ref, w_ref, b_ref, out_ref, m_ref):
        @pl.when(pl.program_id(0) == 0)
        def _():
            m_ref[...] = jnp.dot(
                emb_ref[...], w_ref[...], preferred_element_type=jnp.float32
            )

        acc = jnp.dot(
            counts_ref[...], m_ref[...], preferred_element_type=jnp.float32
        )
        out_ref[...] = jnp.tanh(acc * jnp.float32(inv_n) + b_ref[...])

    return pl.pallas_call(
        body,
        grid=(grid,),
        in_specs=[
            pl.BlockSpec((BT, V), lambda i: (i, 0)),
            pl.BlockSpec((V, E), lambda i: (0, 0)),
            pl.BlockSpec((E, T), lambda i: (0, 0)),
            pl.BlockSpec((1, T), lambda i: (0, 0)),
        ],
        out_specs=pl.BlockSpec((BT, T), lambda i: (i, 0)),
        out_shape=jax.ShapeDtypeStruct((B, T), jnp.float32),
        scratch_shapes=[pltpu.VMEM((V, T), jnp.float32)],
    )(counts, char_emb, W, b2d)


def kernel(ids, char_emb, W, b):
    ids = ids.astype(jnp.int32)
    vocab = char_emb.shape[0]
    counts = _sc_row_histogram(ids, vocab)
    inv_n = 1.0 / ids.shape[1]
    return _tc_project(counts, char_emb, W, b.reshape(1, -1), inv_n)


# speedup vs baseline: 283.9996x; 3.2377x over previous
import functools

import jax
import jax.numpy as jnp
from jax import lax
from jax.experimental import pallas as pl
from jax.experimental.pallas import tpu as pltpu
from jax.experimental.pallas import tpu_sc as plsc

_NC, _NS = 2, 16
_NW = _NC * _NS
_LANES = 16


_UNROLL = 16


_SLAB = 128


def _sc_row_histogram_t(ids_t, vocab):
    S, B = ids_t.shape
    rows_per_w = B // _NW
    slabs = rows_per_w // _SLAB
    gps = _SLAB // _LANES
    mesh = plsc.VectorSubcoreMesh(core_axis_name="c", subcore_axis_name="s")

    ngroups = slabs * gps
    _NH = 4

    @functools.partial(
        pl.kernel,
        mesh=mesh,
        out_type=jax.ShapeDtypeStruct((B, vocab), jnp.float32),
        scratch_types=[
            pltpu.VMEM((S, _SLAB), jnp.int32),
            pltpu.VMEM((S, _SLAB), jnp.int32),
            pltpu.VMEM((_LANES, vocab), jnp.float32),
            pltpu.VMEM((_LANES, vocab), jnp.float32),
            pltpu.VMEM((_LANES, vocab), jnp.float32),
            pltpu.VMEM((_LANES, vocab), jnp.float32),
            pltpu.VMEM_SHARED((_LANES, vocab), jnp.float32),
            pltpu.SemaphoreType.DMA,
            pltpu.SemaphoreType.DMA,
            pltpu.SemaphoreType.DMA,
            pltpu.SemaphoreType.DMA,
            pltpu.SemaphoreType.DMA,
            pltpu.SemaphoreType.DMA,
            pltpu.SemaphoreType.DMA,
            pltpu.SemaphoreType.DMA,
            pltpu.SemaphoreType.DMA,
            pltpu.SemaphoreType.DMA,
        ],
        compiler_params=pltpu.CompilerParams(
            use_tc_tiling_on_sc=True, needs_layout_passes=False
        ),
    )
    def hist_kernel(ids_hbm, counts_hbm, slab_v0, slab_v1,
                    h0, h1, h2, h3, zshared,
                    sem_s0, sem_s1, sem_o0, sem_o1, sem_o2, sem_o3,
                    sem_z0, sem_z1, sem_z2, sem_z3):
        wid = lax.axis_index("s") * _NC + lax.axis_index("c")
        lane = lax.iota(jnp.int32, _LANES)
        ones = jnp.ones((_LANES,), jnp.float32)
        zeros = jnp.zeros((_LANES,), jnp.float32)
        slab_bufs = [(slab_v0, sem_s0), (slab_v1, sem_s1)]
        hist = [h0, h1, h2, h3]
        sem_o = [sem_o0, sem_o1, sem_o2, sem_o3]
        sem_z = [sem_z0, sem_z1, sem_z2, sem_z3]

        def slab_slice(s):
            base = wid * rows_per_w + s * _SLAB
            return ids_hbm.at[:, pl.ds(base, _SLAB)]

        def out_slice(g):
            base = wid * rows_per_w + g * _LANES
            return counts_hbm.at[pl.ds(base, _LANES), :]

        @pl.when(lax.axis_index("s") == 0)
        def _():
            for i in range(_LANES):
                for c in range(vocab // _LANES):
                    h0[i, pl.ds(c * _LANES, _LANES)] = zeros
            pltpu.sync_copy(h0, zshared)

        plsc.subcore_barrier()

        pltpu.async_copy(slab_slice(0), slab_v0, sem_s0)
        pltpu.async_copy(slab_slice(1), slab_v1, sem_s1)
        for u in range(_NH):
            pltpu.async_copy(zshared, hist[u], sem_z[u])

        def group(g, slab_v):
            u = g % _NH
            goff = (g % gps) * _LANES
            pltpu.make_async_copy(zshared, hist[u], sem_z[u]).wait()

            @plsc.parallel_loop(0, S, unroll=_UNROLL)
            def char_body(j):
                idv = slab_v[j, pl.ds(goff, _LANES)]
                plsc.addupdate_scatter(hist[u], [lane, idv], ones)

            pltpu.async_copy(hist[u], out_slice(g), sem_o[u])
            g2 = g + 2
            if g2 >= _NH and g2 < ngroups:
                u2 = g2 % _NH
                pltpu.make_async_copy(hist[u2], out_slice(g), sem_o[u2]).wait()
                pltpu.async_copy(zshared, hist[u2], sem_z[u2])

        for s in range(slabs):
            slab_v, sem_s = slab_bufs[s % 2]
            pltpu.make_async_copy(slab_slice(s), slab_v, sem_s).wait()
            for gl in range(gps):
                group(s * gps + gl, slab_v)
            if s + 2 < slabs:
                pltpu.async_copy(slab_slice(s + 2), slab_v, sem_s)

        for g in range(ngroups - _NH, ngroups):
            u = g % _NH
            pltpu.make_async_copy(hist[u], out_slice(g), sem_o[u]).wait()

    return hist_kernel(ids_t)


def _tc_project(counts, char_emb, W, b2d, inv_n):
    B, V = counts.shape
    E = char_emb.shape[1]
    T = W.shape[1]
    BT = 8192
    grid = B // BT

    def body(counts_ref, emb_ref, w_ref, b_ref, out_ref, m_ref):
        @pl.when(pl.program_id(0) == 0)
        def _():
            m_ref[...] = jnp.dot(
                emb_ref[...], w_ref[...], preferred_element_type=jnp.float32
            )

        acc = jnp.dot(
            counts_ref[...], m_ref[...], preferred_element_type=jnp.float32
        )
        out_ref[...] = jnp.tanh(acc * jnp.float32(inv_n) + b_ref[...])

    return pl.pallas_call(
        body,
        grid=(grid,),
        in_specs=[
            pl.BlockSpec((BT, V), lambda i: (i, 0)),
            pl.BlockSpec((V, E), lambda i: (0, 0)),
            pl.BlockSpec((E, T), lambda i: (0, 0)),
            pl.BlockSpec((1, T), lambda i: (0, 0)),
        ],
        out_specs=pl.BlockSpec((BT, T), lambda i: (i, 0)),
        out_shape=jax.ShapeDtypeStruct((B, T), jnp.float32),
        scratch_shapes=[pltpu.VMEM((V, T), jnp.float32)],
    )(counts, char_emb, W, b2d)


def kernel(ids, char_emb, W, b):
    ids = ids.astype(jnp.int32)
    vocab = char_emb.shape[0]
    inv_n = 1.0 / ids.shape[1]
    counts = _sc_row_histogram_t(ids.T, vocab)
    return _tc_project(counts, char_emb, W, b.reshape(1, -1), inv_n)
